# baseline (speedup 1.0000x reference)
import functools

import jax
import jax.numpy as jnp
from jax.experimental import pallas as pl

_HIDDEN = 4096
_EXPERTS = 128
_TOPK = 8
_TOKENS = 8192
_TB = 256


def _router_block(hs_ref, w_ref, b_ref, idx_ref, wgt_ref):
    logits = jnp.dot(hs_ref[...], w_ref[...], preferred_element_type=jnp.float32)
    scores = jax.nn.sigmoid(logits)
    vals = scores + b_ref[...]
    lanef = jax.lax.broadcasted_iota(jnp.int32, (_TB, _EXPERTS), 1).astype(
        jnp.float32
    )
    crow = lanef + b_ref[...]
    idx_cols = []
    w_cols = []
    for _ in range(_TOPK):
        m = jnp.max(vals, axis=1, keepdims=True)
        eq = vals == m
        c = jnp.min(jnp.where(eq, crow, jnp.inf), axis=1, keepdims=True)
        idxf = jnp.floor(c + 0.5)
        w = m - (c - idxf)
        vals = jnp.where(crow == c, -jnp.inf, vals)
        idx_cols.append(idxf)
        w_cols.append(w)
    idxs = jnp.concatenate(idx_cols, axis=1).astype(jnp.int32)
    ws = jnp.concatenate(w_cols, axis=1)
    ws = ws / (jnp.sum(ws, axis=1, keepdims=True) + 1e-20)
    idx_ref[...] = idxs
    wgt_ref[...] = ws


@functools.partial(jax.jit)
def kernel(hidden_states, weight, e_score_correction_bias):
    hs = hidden_states.reshape(-1, _HIDDEN)
    wt = weight.astype(jnp.float32).T
    bias = e_score_correction_bias.reshape(1, _EXPERTS)
    grid = (_TOKENS // _TB,)
    idxs, ws = pl.pallas_call(
        _router_block,
        grid=grid,
        in_specs=[
            pl.BlockSpec((_TB, _HIDDEN), lambda i: (i, 0)),
            pl.BlockSpec((_HIDDEN, _EXPERTS), lambda i: (0, 0)),
            pl.BlockSpec((1, _EXPERTS), lambda i: (0, 0)),
        ],
        out_specs=[
            pl.BlockSpec((_TB, _TOPK), lambda i: (i, 0)),
            pl.BlockSpec((_TB, _TOPK), lambda i: (i, 0)),
        ],
        out_shape=[
            jax.ShapeDtypeStruct((_TOKENS, _TOPK), jnp.int32),
            jax.ShapeDtypeStruct((_TOKENS, _TOPK), jnp.float32),
        ],
    )(hs, wt, bias)
    return idxs, ws


# speedup vs baseline: 1.3953x; 1.3953x over previous
import functools

import jax
import jax.numpy as jnp
from jax.experimental import pallas as pl

_HIDDEN = 4096
_EXPERTS = 128
_TOPK = 8
_TOKENS = 8192
_TB = 1024


def _router_block(hs_ref, w_ref, b_ref, idx_ref, wgt_ref):
    logits = jnp.dot(hs_ref[...], w_ref[...], preferred_element_type=jnp.float32)
    scores = jax.nn.sigmoid(logits)
    vals = scores + b_ref[...]
    lanef = jax.lax.broadcasted_iota(jnp.int32, (_TB, _EXPERTS), 1).astype(
        jnp.float32
    )
    crow = lanef + b_ref[...]
    idx_cols = []
    w_cols = []
    for _ in range(_TOPK):
        m = jnp.max(vals, axis=1, keepdims=True)
        eq = vals == m
        c = jnp.min(jnp.where(eq, crow, jnp.inf), axis=1, keepdims=True)
        idxf = jnp.floor(c + 0.5)
        w = m - (c - idxf)
        vals = jnp.where(crow == c, -jnp.inf, vals)
        idx_cols.append(idxf)
        w_cols.append(w)
    idxs = jnp.concatenate(idx_cols, axis=1).astype(jnp.int32)
    ws = jnp.concatenate(w_cols, axis=1)
    ws = ws / (jnp.sum(ws, axis=1, keepdims=True) + 1e-20)
    idx_ref[...] = idxs
    wgt_ref[...] = ws


@functools.partial(jax.jit)
def kernel(hidden_states, weight, e_score_correction_bias):
    hs = hidden_states.reshape(-1, _HIDDEN)
    wt = weight.astype(jnp.float32).T
    bias = e_score_correction_bias.reshape(1, _EXPERTS)
    grid = (_TOKENS // _TB,)
    idxs, ws = pl.pallas_call(
        _router_block,
        grid=grid,
        in_specs=[
            pl.BlockSpec((_TB, _HIDDEN), lambda i: (i, 0)),
            pl.BlockSpec((_HIDDEN, _EXPERTS), lambda i: (0, 0)),
            pl.BlockSpec((1, _EXPERTS), lambda i: (0, 0)),
        ],
        out_specs=[
            pl.BlockSpec((_TB, _TOPK), lambda i: (i, 0)),
            pl.BlockSpec((_TB, _TOPK), lambda i: (i, 0)),
        ],
        out_shape=[
            jax.ShapeDtypeStruct((_TOKENS, _TOPK), jnp.int32),
            jax.ShapeDtypeStruct((_TOKENS, _TOPK), jnp.float32),
        ],
    )(hs, wt, bias)
    return idxs, ws
